# Initial kernel scaffold; baseline (speedup 1.0000x reference)
import jax
import jax.numpy as jnp
from jax.experimental import pallas as pl


def kernel(x, edge_index, W1, b1, W2, b2):
    raise NotImplementedError("write your pallas kernel here")


# speedup vs baseline: 32.7515x; 32.7515x over previous
import functools

import jax
import jax.numpy as jnp
from jax import lax
from jax.experimental import pallas as pl
from jax.experimental.pallas import tpu as pltpu
from jax.experimental.pallas import tpu_sc as plsc

F32 = jnp.float32

NC = 2
NS = 16
NW = NC * NS
CHUNK = 128


def _mesh():
    return plsc.VectorSubcoreMesh(core_axis_name="c", subcore_axis_name="s")


def _make_deg_kernel(npad, nchunk):
    rows = npad // NS

    @functools.partial(
        pl.kernel,
        out_type=jax.ShapeDtypeStruct((NC, npad), F32),
        mesh=_mesh(),
        compiler_params=pltpu.CompilerParams(use_tc_tiling_on_sc=False),
        scratch_types=[
            pltpu.VMEM((nchunk, CHUNK), jnp.int32),
            pltpu.VMEM((CHUNK,), F32),
            pltpu.VMEM_SHARED((npad,), F32),
        ],
    )
    def deg_kernel(dst_hbm, ones_hbm, zhist_hbm, deg_hbm, idx_v, ones_v,
                   hist_sp):
        cid = lax.axis_index("c")
        sid = lax.axis_index("s")
        wid = cid * NS + sid
        base = sid * rows
        pltpu.sync_copy(zhist_hbm.at[pl.ds(base, rows)],
                        hist_sp.at[pl.ds(base, rows)])
        pltpu.sync_copy(dst_hbm.at[wid], idx_v)
        pltpu.sync_copy(ones_hbm, ones_v)
        plsc.subcore_barrier()

        def body(j, _):
            pltpu.sync_copy(ones_v, hist_sp.at[idx_v.at[j]], add=True)
            return 0

        lax.fori_loop(0, nchunk, body, 0)
        plsc.subcore_barrier()
        pltpu.sync_copy(hist_sp.at[pl.ds(base, rows)],
                        deg_hbm.at[cid, pl.ds(base, rows)])

    return deg_kernel


def _make_mp_kernel(npad, h, nchunk):
    rows = npad // NS

    @functools.partial(
        pl.kernel,
        out_type=jax.ShapeDtypeStruct((NC, npad, h), F32),
        mesh=_mesh(),
        compiler_params=pltpu.CompilerParams(use_tc_tiling_on_sc=False),
        scratch_types=[
            pltpu.VMEM((nchunk, CHUNK), jnp.int32),
            pltpu.VMEM((nchunk, CHUNK), jnp.int32),
            pltpu.VMEM((CHUNK, h), F32),
            pltpu.VMEM_SHARED((npad, h), F32),
            pltpu.SemaphoreType.DMA,
        ],
    )
    def mp_kernel(y_hbm, src_hbm, dst_hbm, zrows_hbm, parts_hbm,
                  src_v, dst_v, rows_v, accum, sem):
        cid = lax.axis_index("c")
        sid = lax.axis_index("s")
        wid = cid * NS + sid
        base = sid * rows
        pltpu.sync_copy(zrows_hbm, accum.at[pl.ds(base, rows)])
        pltpu.sync_copy(src_hbm.at[wid], src_v)
        pltpu.sync_copy(dst_hbm.at[wid], dst_v)
        plsc.subcore_barrier()

        def body(j, _):
            pltpu.async_copy(y_hbm.at[src_v.at[j]], rows_v, sem).wait()
            pltpu.sync_copy(rows_v, accum.at[dst_v.at[j]], add=True)
            return 0

        lax.fori_loop(0, nchunk, body, 0)
        plsc.subcore_barrier()
        pltpu.sync_copy(accum.at[pl.ds(base, rows)],
                        parts_hbm.at[cid, pl.ds(base, rows)])

    return mp_kernel


def _tc1_body(x_ref, w_ref, deg_ref, y_ref, dinv_ref):
    d = deg_ref[0] + deg_ref[1] + 1.0
    dinv = lax.rsqrt(d)
    xw = jnp.dot(x_ref[...], w_ref[...], preferred_element_type=F32)
    y_ref[...] = xw * dinv
    dinv_ref[...] = dinv


def _tc2_body(p_ref, y1_ref, dinv_ref, b1_ref, w2_ref, y2_ref):
    dinv = dinv_ref[...]
    s = p_ref[0] + p_ref[1] + y1_ref[...]
    hh = jnp.maximum(dinv * s + b1_ref[...], 0.0)
    y2_ref[...] = dinv * jnp.dot(hh, w2_ref[...], preferred_element_type=F32)


def _tc3_body(p_ref, y2_ref, dinv_ref, b2_ref, o_ref):
    o = dinv_ref[...] * (p_ref[0] + p_ref[1] + y2_ref[...]) + b2_ref[...]
    m = jnp.max(o, axis=1, keepdims=True)
    e = jnp.exp(o - m)
    s = jnp.sum(e, axis=1, keepdims=True)
    o_ref[...] = o - m - jnp.log(s)


def kernel(x, edge_index, W1, b1, W2, b2):
    n, d_feat = x.shape
    e = edge_index.shape[1]
    h1 = W1.shape[1]
    h2 = W2.shape[1]

    npad = ((n + NS * 16) // (NS * 16)) * (NS * 16)
    nchunk = -(-e // (NW * CHUNK))
    epad = nchunk * NW * CHUNK
    epw = epad // NW

    pad = epad - e
    src_p = jnp.concatenate([edge_index[0], jnp.zeros((pad,), jnp.int32)])
    dst_p = jnp.concatenate(
        [edge_index[1], jnp.full((pad,), n, jnp.int32)])
    src_r = src_p.reshape(NW, nchunk, CHUNK)
    dst_r = dst_p.reshape(NW, nchunk, CHUNK)
    x_p = jnp.pad(x, ((0, npad - n), (0, 0)))
    zhist = jnp.zeros((npad,), F32)
    zrows = jnp.zeros((npad // NS, h1), F32)
    ones_c = jnp.ones((CHUNK,), F32)

    deg_kernel = _make_deg_kernel(npad, nchunk)
    mp1 = _make_mp_kernel(npad, h1, nchunk)

    degp = deg_kernel(dst_r, ones_c, zhist)

    y1, dinv = pl.pallas_call(
        _tc1_body,
        out_shape=(
            jax.ShapeDtypeStruct((npad, h1), F32),
            jax.ShapeDtypeStruct((npad, 1), F32),
        ),
    )(x_p, W1, degp.reshape(NC, npad, 1))

    p1 = mp1(y1, src_r, dst_r, zrows)

    y2 = pl.pallas_call(
        _tc2_body,
        out_shape=jax.ShapeDtypeStruct((npad, h2), F32),
    )(p1, y1, dinv, b1.reshape(1, h1), W2)

    if h2 != h1:
        mp2 = _make_mp_kernel(npad, h2, nchunk)
        zrows2 = jnp.zeros((npad // NS, h2), F32)
    else:
        mp2, zrows2 = mp1, zrows
    p2 = mp2(y2, src_r, dst_r, zrows2)

    out = pl.pallas_call(
        _tc3_body,
        out_shape=jax.ShapeDtypeStruct((npad, h2), F32),
    )(p2, y2, dinv, b2.reshape(1, h2))

    return out[:n]


# speedup vs baseline: 37.3598x; 1.1407x over previous
import functools

import jax
import jax.numpy as jnp
from jax import lax
from jax.experimental import pallas as pl
from jax.experimental.pallas import tpu as pltpu
from jax.experimental.pallas import tpu_sc as plsc

F32 = jnp.float32

NC = 2
NS = 16
NW = NC * NS
CHUNK = 128


def _mesh():
    return plsc.VectorSubcoreMesh(core_axis_name="c", subcore_axis_name="s")


def _make_deg_kernel(npad, nchunk):
    rows = npad // NS

    @functools.partial(
        pl.kernel,
        out_type=jax.ShapeDtypeStruct((NC, npad), F32),
        mesh=_mesh(),
        compiler_params=pltpu.CompilerParams(use_tc_tiling_on_sc=False),
        scratch_types=[
            pltpu.VMEM((nchunk, CHUNK), jnp.int32),
            pltpu.VMEM((CHUNK,), F32),
            pltpu.VMEM_SHARED((npad,), F32),
        ],
    )
    def deg_kernel(dst_hbm, ones_hbm, zhist_hbm, deg_hbm, idx_v, ones_v,
                   hist_sp):
        cid = lax.axis_index("c")
        sid = lax.axis_index("s")
        wid = cid * NS + sid
        base = sid * rows
        pltpu.sync_copy(zhist_hbm.at[pl.ds(base, rows)],
                        hist_sp.at[pl.ds(base, rows)])
        pltpu.sync_copy(dst_hbm.at[wid], idx_v)
        pltpu.sync_copy(ones_hbm, ones_v)
        plsc.subcore_barrier()

        def body(j, _):
            pltpu.sync_copy(ones_v, hist_sp.at[idx_v.at[j]], add=True)
            return 0

        lax.fori_loop(0, nchunk, body, 0)
        plsc.subcore_barrier()
        pltpu.sync_copy(hist_sp.at[pl.ds(base, rows)],
                        deg_hbm.at[cid, pl.ds(base, rows)])

    return deg_kernel


GRP = 4


def _make_mp_kernel(npad, h, nchunk):
    rows = npad // NS
    ngrp = nchunk // GRP

    @functools.partial(
        pl.kernel,
        out_type=jax.ShapeDtypeStruct((NC, npad, h), F32),
        mesh=_mesh(),
        compiler_params=pltpu.CompilerParams(use_tc_tiling_on_sc=False),
        scratch_types=[
            pltpu.VMEM((nchunk, CHUNK), jnp.int32),
            pltpu.VMEM((nchunk, CHUNK), jnp.int32),
            pltpu.VMEM((GRP, CHUNK, h), F32),
            pltpu.VMEM_SHARED((npad, h), F32),
            pltpu.SemaphoreType.DMA,
            pltpu.SemaphoreType.DMA,
        ],
    )
    def mp_kernel(y_hbm, src_hbm, dst_hbm, zrows_hbm, parts_hbm,
                  src_v, dst_v, bufs, accum, gsem, ssem):
        cid = lax.axis_index("c")
        sid = lax.axis_index("s")
        wid = cid * NS + sid
        base = sid * rows
        pltpu.sync_copy(zrows_hbm, accum.at[pl.ds(base, rows)])
        pltpu.sync_copy(src_hbm.at[wid], src_v)
        pltpu.sync_copy(dst_hbm.at[wid], dst_v)
        plsc.subcore_barrier()

        def drain_scatters():
            for k in range(GRP):
                pltpu.make_async_copy(
                    bufs.at[k], accum.at[pl.ds(0, CHUNK)], ssem).wait()

        def body(g, _):
            @pl.when(g > 0)
            def _():
                drain_scatters()

            gds = [
                pltpu.async_copy(
                    y_hbm.at[src_v.at[g * GRP + k]], bufs.at[k], gsem)
                for k in range(GRP)
            ]
            for d in gds:
                d.wait()
            for k in range(GRP):
                pltpu.async_copy(
                    bufs.at[k], accum.at[dst_v.at[g * GRP + k]], ssem,
                    add=True)
            return 0

        lax.fori_loop(0, ngrp, body, 0)
        drain_scatters()
        plsc.subcore_barrier()
        pltpu.sync_copy(accum.at[pl.ds(base, rows)],
                        parts_hbm.at[cid, pl.ds(base, rows)])

    return mp_kernel


def _tc1_body(x_ref, w_ref, deg_ref, y_ref, dinv_ref):
    d = deg_ref[0] + deg_ref[1] + 1.0
    dinv = lax.rsqrt(d)
    xw = jnp.dot(x_ref[...], w_ref[...], preferred_element_type=F32)
    y_ref[...] = xw * dinv
    dinv_ref[...] = dinv


def _tc2_body(p_ref, y1_ref, dinv_ref, b1_ref, w2_ref, y2_ref):
    dinv = dinv_ref[...]
    s = p_ref[0] + p_ref[1] + y1_ref[...]
    hh = jnp.maximum(dinv * s + b1_ref[...], 0.0)
    y2_ref[...] = dinv * jnp.dot(hh, w2_ref[...], preferred_element_type=F32)


def _tc3_body(p_ref, y2_ref, dinv_ref, b2_ref, o_ref):
    o = dinv_ref[...] * (p_ref[0] + p_ref[1] + y2_ref[...]) + b2_ref[...]
    m = jnp.max(o, axis=1, keepdims=True)
    e = jnp.exp(o - m)
    s = jnp.sum(e, axis=1, keepdims=True)
    o_ref[...] = o - m - jnp.log(s)


def kernel(x, edge_index, W1, b1, W2, b2):
    n, d_feat = x.shape
    e = edge_index.shape[1]
    h1 = W1.shape[1]
    h2 = W2.shape[1]

    npad = ((n + NS * 16) // (NS * 16)) * (NS * 16)
    nchunk = (-(-e // (NW * CHUNK * GRP))) * GRP
    epad = nchunk * NW * CHUNK
    epw = epad // NW

    pad = epad - e
    src_p = jnp.concatenate([edge_index[0], jnp.zeros((pad,), jnp.int32)])
    dst_p = jnp.concatenate(
        [edge_index[1], jnp.full((pad,), n, jnp.int32)])
    src_r = src_p.reshape(NW, nchunk, CHUNK)
    dst_r = dst_p.reshape(NW, nchunk, CHUNK)
    x_p = jnp.pad(x, ((0, npad - n), (0, 0)))
    zhist = jnp.zeros((npad,), F32)
    zrows = jnp.zeros((npad // NS, h1), F32)
    ones_c = jnp.ones((CHUNK,), F32)

    deg_kernel = _make_deg_kernel(npad, nchunk)
    mp1 = _make_mp_kernel(npad, h1, nchunk)

    degp = deg_kernel(dst_r, ones_c, zhist)

    y1, dinv = pl.pallas_call(
        _tc1_body,
        out_shape=(
            jax.ShapeDtypeStruct((npad, h1), F32),
            jax.ShapeDtypeStruct((npad, 1), F32),
        ),
    )(x_p, W1, degp.reshape(NC, npad, 1))

    p1 = mp1(y1, src_r, dst_r, zrows)

    y2 = pl.pallas_call(
        _tc2_body,
        out_shape=jax.ShapeDtypeStruct((npad, h2), F32),
    )(p1, y1, dinv, b1.reshape(1, h1), W2)

    if h2 != h1:
        mp2 = _make_mp_kernel(npad, h2, nchunk)
        zrows2 = jnp.zeros((npad // NS, h2), F32)
    else:
        mp2, zrows2 = mp1, zrows
    p2 = mp2(y2, src_r, dst_r, zrows2)

    out = pl.pallas_call(
        _tc3_body,
        out_shape=jax.ShapeDtypeStruct((npad, h2), F32),
    )(p2, y2, dinv, b2.reshape(1, h2))

    return out[:n]


# speedup vs baseline: 42.1746x; 1.1289x over previous
import functools

import jax
import jax.numpy as jnp
from jax import lax
from jax.experimental import pallas as pl
from jax.experimental.pallas import tpu as pltpu
from jax.experimental.pallas import tpu_sc as plsc

F32 = jnp.float32

NC = 2
NS = 16
NW = NC * NS
CHUNK = 128


def _mesh():
    return plsc.VectorSubcoreMesh(core_axis_name="c", subcore_axis_name="s")


def _make_deg_kernel(npad, nchunk):
    rows = npad // NS

    @functools.partial(
        pl.kernel,
        out_type=jax.ShapeDtypeStruct((NC, npad), F32),
        mesh=_mesh(),
        compiler_params=pltpu.CompilerParams(use_tc_tiling_on_sc=False),
        scratch_types=[
            pltpu.VMEM((nchunk, CHUNK), jnp.int32),
            pltpu.VMEM((CHUNK,), F32),
            pltpu.VMEM_SHARED((npad,), F32),
        ],
    )
    def deg_kernel(dst_hbm, ones_hbm, zhist_hbm, deg_hbm, idx_v, ones_v,
                   hist_sp):
        cid = lax.axis_index("c")
        sid = lax.axis_index("s")
        wid = cid * NS + sid
        base = sid * rows
        pltpu.sync_copy(zhist_hbm.at[pl.ds(base, rows)],
                        hist_sp.at[pl.ds(base, rows)])
        pltpu.sync_copy(dst_hbm.at[wid], idx_v)
        pltpu.sync_copy(ones_hbm, ones_v)
        plsc.subcore_barrier()

        def body(j, _):
            pltpu.sync_copy(ones_v, hist_sp.at[idx_v.at[j]], add=True)
            return 0

        lax.fori_loop(0, nchunk, body, 0)
        plsc.subcore_barrier()
        pltpu.sync_copy(hist_sp.at[pl.ds(base, rows)],
                        deg_hbm.at[cid, pl.ds(base, rows)])

    return deg_kernel


GRP = 4


def _make_mp_kernel(npad, h, nchunk):
    rows = npad // NS
    ngrp = nchunk // GRP
    NB = 4

    @functools.partial(
        pl.kernel,
        out_type=jax.ShapeDtypeStruct((NC, npad, h), F32),
        mesh=_mesh(),
        compiler_params=pltpu.CompilerParams(use_tc_tiling_on_sc=False),
        scratch_types=[
            pltpu.VMEM((nchunk, CHUNK), jnp.int32),
            pltpu.VMEM((nchunk, CHUNK), jnp.int32),
            pltpu.VMEM((NB * GRP, CHUNK, h), F32),
            pltpu.VMEM_SHARED((npad, h), F32),
            pltpu.SemaphoreType.DMA((NB,)),
            pltpu.SemaphoreType.DMA((NB,)),
        ],
    )
    def mp_kernel(y_hbm, src_hbm, dst_hbm, zrows_hbm, parts_hbm,
                  src_v, dst_v, bufs, accum, gsem, ssem):
        cid = lax.axis_index("c")
        sid = lax.axis_index("s")
        wid = cid * NS + sid
        base = sid * rows
        pltpu.sync_copy(zrows_hbm, accum.at[pl.ds(base, rows)])
        pltpu.sync_copy(src_hbm.at[wid], src_v)
        pltpu.sync_copy(dst_hbm.at[wid], dst_v)
        plsc.subcore_barrier()

        def fire_gathers(g, b):
            for k in range(GRP):
                pltpu.async_copy(
                    y_hbm.at[src_v.at[g * GRP + k]],
                    bufs.at[b * GRP + k], gsem.at[b])

        def drain(sem, b):
            for k in range(GRP):
                pltpu.make_async_copy(
                    bufs.at[k], accum.at[pl.ds(0, CHUNK)], sem.at[b]).wait()

        fire_gathers(0, 0)

        def body(g, _):
            b = lax.rem(g, NB)
            nb = lax.rem(g + 1, NB)

            @pl.when(jnp.logical_and(g + 1 < ngrp, g + 1 >= NB))
            def _():
                drain(ssem, nb)

            @pl.when(g + 1 < ngrp)
            def _():
                fire_gathers(g + 1, nb)

            drain(gsem, b)
            for k in range(GRP):
                pltpu.async_copy(
                    bufs.at[b * GRP + k],
                    accum.at[dst_v.at[g * GRP + k]], ssem.at[b], add=True)
            return 0

        lax.fori_loop(0, ngrp, body, 0)
        for j in range(max(ngrp - NB, 0), ngrp):
            drain(ssem, j % NB)
        plsc.subcore_barrier()
        pltpu.sync_copy(accum.at[pl.ds(base, rows)],
                        parts_hbm.at[cid, pl.ds(base, rows)])

    return mp_kernel


def _tc1_body(x_ref, w_ref, deg_ref, y_ref, dinv_ref):
    d = deg_ref[0] + deg_ref[1] + 1.0
    dinv = lax.rsqrt(d)
    xw = jnp.dot(x_ref[...], w_ref[...], preferred_element_type=F32)
    y_ref[...] = xw * dinv
    dinv_ref[...] = dinv


def _tc2_body(p_ref, y1_ref, dinv_ref, b1_ref, w2_ref, y2_ref):
    dinv = dinv_ref[...]
    s = p_ref[0] + p_ref[1] + y1_ref[...]
    hh = jnp.maximum(dinv * s + b1_ref[...], 0.0)
    y2_ref[...] = dinv * jnp.dot(hh, w2_ref[...], preferred_element_type=F32)


def _tc3_body(p_ref, y2_ref, dinv_ref, b2_ref, o_ref):
    o = dinv_ref[...] * (p_ref[0] + p_ref[1] + y2_ref[...]) + b2_ref[...]
    m = jnp.max(o, axis=1, keepdims=True)
    e = jnp.exp(o - m)
    s = jnp.sum(e, axis=1, keepdims=True)
    o_ref[...] = o - m - jnp.log(s)


def kernel(x, edge_index, W1, b1, W2, b2):
    n, d_feat = x.shape
    e = edge_index.shape[1]
    h1 = W1.shape[1]
    h2 = W2.shape[1]

    npad = ((n + NS * 16) // (NS * 16)) * (NS * 16)
    nchunk = (-(-e // (NW * CHUNK * GRP))) * GRP
    epad = nchunk * NW * CHUNK
    epw = epad // NW

    pad = epad - e
    src_p = jnp.concatenate([edge_index[0], jnp.zeros((pad,), jnp.int32)])
    dst_p = jnp.concatenate(
        [edge_index[1], jnp.full((pad,), n, jnp.int32)])
    src_r = src_p.reshape(NW, nchunk, CHUNK)
    dst_r = dst_p.reshape(NW, nchunk, CHUNK)
    x_p = jnp.pad(x, ((0, npad - n), (0, 0)))
    zhist = jnp.zeros((npad,), F32)
    zrows = jnp.zeros((npad // NS, h1), F32)
    ones_c = jnp.ones((CHUNK,), F32)

    deg_kernel = _make_deg_kernel(npad, nchunk)
    mp1 = _make_mp_kernel(npad, h1, nchunk)

    degp = deg_kernel(dst_r, ones_c, zhist)

    y1, dinv = pl.pallas_call(
        _tc1_body,
        out_shape=(
            jax.ShapeDtypeStruct((npad, h1), F32),
            jax.ShapeDtypeStruct((npad, 1), F32),
        ),
    )(x_p, W1, degp.reshape(NC, npad, 1))

    p1 = mp1(y1, src_r, dst_r, zrows)

    y2 = pl.pallas_call(
        _tc2_body,
        out_shape=jax.ShapeDtypeStruct((npad, h2), F32),
    )(p1, y1, dinv, b1.reshape(1, h1), W2)

    if h2 != h1:
        mp2 = _make_mp_kernel(npad, h2, nchunk)
        zrows2 = jnp.zeros((npad // NS, h2), F32)
    else:
        mp2, zrows2 = mp1, zrows
    p2 = mp2(y2, src_r, dst_r, zrows2)

    out = pl.pallas_call(
        _tc3_body,
        out_shape=jax.ShapeDtypeStruct((npad, h2), F32),
    )(p2, y2, dinv, b2.reshape(1, h2))

    return out[:n]
